# Initial kernel scaffold; baseline (speedup 1.0000x reference)
import jax
import jax.numpy as jnp
from jax.experimental import pallas as pl


def kernel(g1_x, g1_edge_index, g1_batch, g2_x, g2_edge_index, g2_batch, W11, b11, W12, b12, W21, b21, W22, b22):
    raise NotImplementedError("write your pallas kernel here")


# speedup vs baseline: 1.9077x; 1.9077x over previous
import functools

import jax
import jax.numpy as jnp
from jax import lax
from jax.experimental import pallas as pl
from jax.experimental.pallas import tpu as pltpu
from jax.experimental.pallas import tpu_sc as plsc

N = 10000
D = 256
DC = 128
E = 160000
G = 64

NT = 16
EW = 128
EPT = 10240
NWIN = EPT // EW
EPAD = EPT * NT
TRASH = N
ACC_ROWS = 10240

ROWS_PER_TILE_ZERO = ACC_ROWS // NT // 128
OUT_CHUNK = 640
OUT_LAST = N - OUT_CHUNK * (NT - 1)

@functools.cache
def _mesh():
    return plsc.VectorSubcoreMesh(core_axis_name="c", subcore_axis_name="s")


def _sc_agg(x0, x1, srcp, dstp):
    @functools.partial(
        pl.kernel,
        out_type=[jax.ShapeDtypeStruct((N, DC), jnp.float32),
                  jax.ShapeDtypeStruct((N, DC), jnp.float32)],
        mesh=_mesh(),
        scratch_types=[
            pltpu.VMEM((EW,), jnp.int32),
            pltpu.VMEM((EW,), jnp.int32),
            pltpu.VMEM((EW, DC), jnp.float32),
            pltpu.VMEM_SHARED((ACC_ROWS, DC), jnp.float32),
        ],
    )
    def k(x0_hbm, x1_hbm, src_hbm, dst_hbm, o0_hbm, o1_hbm,
          srcw, dstw, rows, acc):
        c = lax.axis_index("c")
        s = lax.axis_index("s")

        zero16 = jnp.zeros((16,), jnp.float32)

        @pl.loop(0, EW)
        def _(r):
            for j in range(DC // 16):
                rows[r, pl.ds(j * 16, 16)] = zero16

        @pl.loop(0, ROWS_PER_TILE_ZERO)
        def _(z):
            row0 = (s * ROWS_PER_TILE_ZERO + z) * 128
            pltpu.sync_copy(rows, acc.at[pl.ds(row0, 128)])

        plsc.subcore_barrier()

        def gather_scatter(x_hbm):
            base = s * EPT

            @pl.loop(0, NWIN)
            def _(w):
                off = base + w * EW
                pltpu.sync_copy(src_hbm.at[pl.ds(off, EW)], srcw)
                pltpu.sync_copy(dst_hbm.at[pl.ds(off, EW)], dstw)
                pltpu.sync_copy(x_hbm.at[srcw], rows)
                pltpu.sync_copy(rows, acc.at[dstw], add=True)

        @pl.when(c == 0)
        def _():
            gather_scatter(x0_hbm)

        @pl.when(c == 1)
        def _():
            gather_scatter(x1_hbm)

        plsc.subcore_barrier()

        row0 = s * OUT_CHUNK
        sl_full = pl.ds(row0, OUT_CHUNK)
        sl_last = pl.ds((NT - 1) * OUT_CHUNK, OUT_LAST)

        def copy_out(o_hbm):
            @pl.when(s < NT - 1)
            def _():
                pltpu.sync_copy(acc.at[sl_full], o_hbm.at[sl_full])

            @pl.when(s == NT - 1)
            def _():
                pltpu.sync_copy(acc.at[sl_last], o_hbm.at[sl_last])

        @pl.when(c == 0)
        def _():
            copy_out(o0_hbm)

        @pl.when(c == 1)
        def _():
            copy_out(o1_hbm)

    return k(x0, x1, srcp, dstp)


_R1 = 1000


def _tc_mlp1(x0, x1, a0, a1, w11, b11, w12, b12):
    def body(x0r, x1r, a0r, a1r, w11r, b11r, w12r, b12r, o0r, o1r):
        t = (x0r[...] + a0r[...]) @ w11r[:DC, :]
        t += (x1r[...] + a1r[...]) @ w11r[DC:, :]
        t = jnp.maximum(t + b11r[...], 0.0)
        u = jnp.maximum(t @ w12r[...] + b12r[...], 0.0)
        o0r[...] = u[:, :DC]
        o1r[...] = u[:, DC:]

    nb = N // _R1
    row_spec = pl.BlockSpec((_R1, DC), lambda i: (i, 0))
    w_spec = pl.BlockSpec((D, D), lambda i: (0, 0))
    b_spec = pl.BlockSpec((1, D), lambda i: (0, 0))
    return pl.pallas_call(
        body,
        grid=(nb,),
        in_specs=[row_spec, row_spec, row_spec, row_spec,
                  w_spec, b_spec, w_spec, b_spec],
        out_specs=[row_spec, row_spec],
        out_shape=[jax.ShapeDtypeStruct((N, DC), jnp.float32),
                   jax.ShapeDtypeStruct((N, DC), jnp.float32)],
    )(x0, x1, a0, a1, w11, b11, w12, b12)


def _tc_pool(h0, h1, a0, a1, batch3, w21, b21, w22, b22):
    nb = N // _R1

    def body(h0r, h1r, a0r, a1r, br, w21r, b21r, w22r, b22r, qr,
             pooled, counts):
        i = pl.program_id(0)

        @pl.when(i == 0)
        def _():
            pooled[...] = jnp.zeros_like(pooled)
            counts[...] = jnp.zeros_like(counts)

        t = (h0r[...] + a0r[...]) @ w21r[:DC, :]
        t += (h1r[...] + a1r[...]) @ w21r[DC:, :]
        t = jnp.maximum(t + b21r[...], 0.0)
        seg = br[0, 0, :]
        onehot = (lax.broadcasted_iota(jnp.int32, (G, _R1), 0)
                  == seg[None, :]).astype(jnp.float32)
        pooled[...] += onehot @ t
        counts[...] += jnp.sum(onehot, axis=1, keepdims=True)

        @pl.when(i == nb - 1)
        def _():
            qr[...] = (pooled[...] @ w22r[...]
                       + counts[:, :1] * b22r[...])

    row_spec = pl.BlockSpec((_R1, DC), lambda i: (i, 0))
    w_spec = pl.BlockSpec((D, D), lambda i: (0, 0))
    b_spec = pl.BlockSpec((1, D), lambda i: (0, 0))
    return pl.pallas_call(
        body,
        grid=(nb,),
        in_specs=[row_spec, row_spec, row_spec, row_spec,
                  pl.BlockSpec((1, 1, _R1), lambda i: (i, 0, 0)),
                  w_spec, b_spec, w_spec, b_spec],
        out_specs=pl.BlockSpec((G, D), lambda i: (0, 0)),
        out_shape=jax.ShapeDtypeStruct((G, D), jnp.float32),
        scratch_shapes=[pltpu.VMEM((G, D), jnp.float32),
                        pltpu.VMEM((G, 128), jnp.float32)],
    )(h0, h1, a0, a1, batch3, w21, b21, w22, b22)


def _tc_final(q1, q2):
    def body(q1r, q2r, outr):
        z = jnp.abs(q1r[...] - q2r[...])
        outr[...] = jnp.exp(-jnp.sum(z, axis=1))[None, :]

    return pl.pallas_call(
        body,
        out_shape=jax.ShapeDtypeStruct((1, G), jnp.float32),
    )(q1, q2)


def _prep_edges(edge_index):
    pad = EPAD - E
    srcp = jnp.concatenate(
        [edge_index[0], jnp.zeros((pad,), jnp.int32)])
    dstp = jnp.concatenate(
        [edge_index[1], jnp.full((pad,), TRASH, jnp.int32)])
    return srcp, dstp


def kernel(g1_x, g1_edge_index, g1_batch, g2_x, g2_edge_index, g2_batch,
           W11, b11, W12, b12, W21, b21, W22, b22):
    x10, x11 = g1_x[:, :DC], g1_x[:, DC:]
    x20, x21 = g2_x[:, :DC], g2_x[:, DC:]
    s1, d1 = _prep_edges(g1_edge_index)
    s2, d2 = _prep_edges(g2_edge_index)
    b11r = b11.reshape(1, D)
    b12r = b12.reshape(1, D)
    b21r = b21.reshape(1, D)
    b22r = b22.reshape(1, D)
    batch1 = g1_batch.reshape(N // _R1, 1, _R1)
    batch2 = g2_batch.reshape(N // _R1, 1, _R1)

    a10, a11 = _sc_agg(x10, x11, s1, d1)
    a20, a21 = _sc_agg(x20, x21, s2, d2)
    h10, h11 = _tc_mlp1(x10, x11, a10, a11, W11, b11r, W12, b12r)
    h20, h21 = _tc_mlp1(x20, x21, a20, a21, W11, b11r, W12, b12r)
    c10, c11 = _sc_agg(h10, h11, s1, d1)
    c20, c21 = _sc_agg(h20, h21, s2, d2)
    q1 = _tc_pool(h10, h11, c10, c11, batch1, W21, b21r, W22, b22r)
    q2 = _tc_pool(h20, h21, c20, c21, batch2, W21, b21r, W22, b22r)
    return _tc_final(q1, q2).reshape(G)


# speedup vs baseline: 2.7273x; 1.4296x over previous
import functools

import jax
import jax.numpy as jnp
from jax import lax
from jax.experimental import pallas as pl
from jax.experimental.pallas import tpu as pltpu
from jax.experimental.pallas import tpu_sc as plsc

N = 10000
D = 256
DC = 128
E = 160000
G = 64

NT = 16
EW = 128
EPT = 10240
NWIN = EPT // EW
EPAD = EPT * NT
TRASH = N
ACC_ROWS = 10112

NZCHUNK = ACC_ROWS // 128
OUT_CHUNK = 640
OUT_LAST = N - OUT_CHUNK * (NT - 1)

@functools.cache
def _mesh():
    return plsc.VectorSubcoreMesh(core_axis_name="c", subcore_axis_name="s")


def _sc_agg(x0, x1, srcp, dstp):
    @functools.partial(
        pl.kernel,
        out_type=[jax.ShapeDtypeStruct((N, DC), jnp.float32),
                  jax.ShapeDtypeStruct((N, DC), jnp.float32)],
        mesh=_mesh(),
        scratch_types=[
            pltpu.VMEM((NWIN, EW), jnp.int32),
            pltpu.VMEM((EW,), jnp.int32),
            pltpu.VMEM((EW,), jnp.int32),
            pltpu.VMEM((EW, DC), jnp.float32),
            pltpu.VMEM((EW, DC), jnp.float32),
            pltpu.VMEM_SHARED((ACC_ROWS, DC), jnp.float32),
            pltpu.SemaphoreType.DMA,
            pltpu.SemaphoreType.DMA,
            pltpu.SemaphoreType.DMA,
            pltpu.SemaphoreType.DMA,
        ],
    )
    def k(x0_hbm, x1_hbm, src_hbm, dst_hbm, o0_hbm, o1_hbm,
          src2d, dstb0, dstb1, rows0, rows1, acc, gsa, gsb, dsa, dsb):
        c = lax.axis_index("c")
        s = lax.axis_index("s")

        pltpu.sync_copy(src_hbm.at[s], src2d)

        zero16 = jnp.zeros((16,), jnp.float32)

        @pl.loop(0, EW)
        def _(r):
            for j in range(DC // 16):
                rows0[r, pl.ds(j * 16, 16)] = zero16

        @pl.loop(0, (NZCHUNK + NT - 1) // NT)
        def _(z):
            chunk = z * NT + s

            @pl.when(chunk < NZCHUNK)
            def _():
                pltpu.sync_copy(rows0, acc.at[pl.ds(chunk * 128, 128)])

        plsc.subcore_barrier()

        mydst = dst_hbm.at[s]

        def gather_scatter(x_hbm):
            dummy = x_hbm.at[pl.ds(0, EW)]
            idummy = mydst.at[0]
            pltpu.async_copy(mydst.at[0], dstb0, dsa)
            pltpu.async_copy(mydst.at[1], dstb1, dsb)
            pltpu.async_copy(x_hbm.at[src2d.at[0]], rows0, gsa)
            pltpu.async_copy(x_hbm.at[src2d.at[1]], rows1, gsb)

            @pl.loop(0, NWIN // 2)
            def _(i):
                w0 = 2 * i
                pltpu.make_async_copy(dummy, rows0, gsa).wait()
                pltpu.make_async_copy(idummy, dstb0, dsa).wait()
                pltpu.sync_copy(rows0, acc.at[dstb0], add=True)

                @pl.when(w0 + 2 < NWIN)
                def _():
                    pltpu.async_copy(x_hbm.at[src2d.at[w0 + 2]], rows0, gsa)
                    pltpu.async_copy(mydst.at[w0 + 2], dstb0, dsa)

                pltpu.make_async_copy(dummy, rows1, gsb).wait()
                pltpu.make_async_copy(idummy, dstb1, dsb).wait()
                pltpu.sync_copy(rows1, acc.at[dstb1], add=True)

                @pl.when(w0 + 3 < NWIN)
                def _():
                    pltpu.async_copy(x_hbm.at[src2d.at[w0 + 3]], rows1, gsb)
                    pltpu.async_copy(mydst.at[w0 + 3], dstb1, dsb)

        @pl.when(c == 0)
        def _():
            gather_scatter(x0_hbm)

        @pl.when(c == 1)
        def _():
            gather_scatter(x1_hbm)

        plsc.subcore_barrier()

        row0 = s * OUT_CHUNK
        sl_full = pl.ds(row0, OUT_CHUNK)
        sl_last = pl.ds((NT - 1) * OUT_CHUNK, OUT_LAST)

        def copy_out(o_hbm):
            @pl.when(s < NT - 1)
            def _():
                pltpu.sync_copy(acc.at[sl_full], o_hbm.at[sl_full])

            @pl.when(s == NT - 1)
            def _():
                pltpu.sync_copy(acc.at[sl_last], o_hbm.at[sl_last])

        @pl.when(c == 0)
        def _():
            copy_out(o0_hbm)

        @pl.when(c == 1)
        def _():
            copy_out(o1_hbm)

    return k(x0, x1, srcp, dstp)


_R1 = 1000


def _tc_mlp1(x0, x1, a0, a1, w11, b11, w12, b12):
    def body(x0r, x1r, a0r, a1r, w11r, b11r, w12r, b12r, o0r, o1r):
        t = (x0r[...] + a0r[...]) @ w11r[:DC, :]
        t += (x1r[...] + a1r[...]) @ w11r[DC:, :]
        t = jnp.maximum(t + b11r[...], 0.0)
        u = jnp.maximum(t @ w12r[...] + b12r[...], 0.0)
        o0r[...] = u[:, :DC]
        o1r[...] = u[:, DC:]

    nb = N // _R1
    row_spec = pl.BlockSpec((_R1, DC), lambda i: (i, 0))
    w_spec = pl.BlockSpec((D, D), lambda i: (0, 0))
    b_spec = pl.BlockSpec((1, D), lambda i: (0, 0))
    return pl.pallas_call(
        body,
        grid=(nb,),
        in_specs=[row_spec, row_spec, row_spec, row_spec,
                  w_spec, b_spec, w_spec, b_spec],
        out_specs=[row_spec, row_spec],
        out_shape=[jax.ShapeDtypeStruct((N, DC), jnp.float32),
                   jax.ShapeDtypeStruct((N, DC), jnp.float32)],
    )(x0, x1, a0, a1, w11, b11, w12, b12)


def _tc_pool(h0, h1, a0, a1, batch3, w21, b21, w22, b22):
    nb = N // _R1

    def body(h0r, h1r, a0r, a1r, br, w21r, b21r, w22r, b22r, qr,
             pooled, counts):
        i = pl.program_id(0)

        @pl.when(i == 0)
        def _():
            pooled[...] = jnp.zeros_like(pooled)
            counts[...] = jnp.zeros_like(counts)

        t = (h0r[...] + a0r[...]) @ w21r[:DC, :]
        t += (h1r[...] + a1r[...]) @ w21r[DC:, :]
        t = jnp.maximum(t + b21r[...], 0.0)
        seg = br[0, 0, :]
        onehot = (lax.broadcasted_iota(jnp.int32, (G, _R1), 0)
                  == seg[None, :]).astype(jnp.float32)
        pooled[...] += onehot @ t
        counts[...] += jnp.sum(onehot, axis=1, keepdims=True)

        @pl.when(i == nb - 1)
        def _():
            qr[...] = (pooled[...] @ w22r[...]
                       + counts[:, :1] * b22r[...])

    row_spec = pl.BlockSpec((_R1, DC), lambda i: (i, 0))
    w_spec = pl.BlockSpec((D, D), lambda i: (0, 0))
    b_spec = pl.BlockSpec((1, D), lambda i: (0, 0))
    return pl.pallas_call(
        body,
        grid=(nb,),
        in_specs=[row_spec, row_spec, row_spec, row_spec,
                  pl.BlockSpec((1, 1, _R1), lambda i: (i, 0, 0)),
                  w_spec, b_spec, w_spec, b_spec],
        out_specs=pl.BlockSpec((G, D), lambda i: (0, 0)),
        out_shape=jax.ShapeDtypeStruct((G, D), jnp.float32),
        scratch_shapes=[pltpu.VMEM((G, D), jnp.float32),
                        pltpu.VMEM((G, 128), jnp.float32)],
    )(h0, h1, a0, a1, batch3, w21, b21, w22, b22)


def _tc_final(q1, q2):
    def body(q1r, q2r, outr):
        z = jnp.abs(q1r[...] - q2r[...])
        outr[...] = jnp.exp(-jnp.sum(z, axis=1))[None, :]

    return pl.pallas_call(
        body,
        out_shape=jax.ShapeDtypeStruct((1, G), jnp.float32),
    )(q1, q2)


def _prep_edges(edge_index):
    pad = EPAD - E
    srcp = jnp.concatenate(
        [edge_index[0], jnp.zeros((pad,), jnp.int32)]).reshape(NT, NWIN, EW)
    dstp = jnp.concatenate(
        [edge_index[1], jnp.full((pad,), TRASH, jnp.int32)]).reshape(NT, NWIN, EW)
    return srcp, dstp


def kernel(g1_x, g1_edge_index, g1_batch, g2_x, g2_edge_index, g2_batch,
           W11, b11, W12, b12, W21, b21, W22, b22):
    x10, x11 = g1_x[:, :DC], g1_x[:, DC:]
    x20, x21 = g2_x[:, :DC], g2_x[:, DC:]
    s1, d1 = _prep_edges(g1_edge_index)
    s2, d2 = _prep_edges(g2_edge_index)
    b11r = b11.reshape(1, D)
    b12r = b12.reshape(1, D)
    b21r = b21.reshape(1, D)
    b22r = b22.reshape(1, D)
    batch1 = g1_batch.reshape(N // _R1, 1, _R1)
    batch2 = g2_batch.reshape(N // _R1, 1, _R1)

    a10, a11 = _sc_agg(x10, x11, s1, d1)
    a20, a21 = _sc_agg(x20, x21, s2, d2)
    h10, h11 = _tc_mlp1(x10, x11, a10, a11, W11, b11r, W12, b12r)
    h20, h21 = _tc_mlp1(x20, x21, a20, a21, W11, b11r, W12, b12r)
    c10, c11 = _sc_agg(h10, h11, s1, d1)
    c20, c21 = _sc_agg(h20, h21, s2, d2)
    q1 = _tc_pool(h10, h11, c10, c11, batch1, W21, b21r, W22, b22r)
    q2 = _tc_pool(h20, h21, c20, c21, batch2, W21, b21r, W22, b22r)
    return _tc_final(q1, q2).reshape(G)
